# baseline (speedup 1.0000x reference)
import functools

import jax
import jax.numpy as jnp
from jax.experimental import pallas as pl
from jax.experimental.pallas import tpu as pltpu

_DEPTH = 8
_BLK = 32
_CHUNK = 16


def _score_kernel(hproj_ref, w_ref, v_ref, o_hbm, out_ref, buf, sems,
                  *, s_half):
    p = pl.program_id(0)
    base = p * s_half
    n_blk = s_half // _BLK

    def start(i):
        slot = jax.lax.rem(i, _DEPTH)
        pltpu.make_async_copy(
            o_hbm.at[pl.ds(base + i * _BLK, _BLK)],
            buf.at[slot], sems.at[slot]).start()

    for i in range(min(_DEPTH - 1, n_blk)):
        start(i)

    def body(i, _):
        slot = jax.lax.rem(i, _DEPTH)
        row0 = pl.multiple_of(i * _BLK, _BLK)
        pltpu.make_async_copy(buf.at[slot], buf.at[slot],
                              sems.at[slot]).wait()
        for c in range(_BLK // _CHUNK):
            o = buf[slot, pl.ds(c * _CHUNK, _CHUNK), :, :]
            cs, b, h = o.shape
            lhs = o.reshape(cs * b, h).astype(jnp.bfloat16)
            oproj = jnp.dot(lhs, w_ref[...],
                            preferred_element_type=jnp.float32)
            t = jnp.tanh(oproj.reshape(cs, b, h) + hproj_ref[...][None, :, :])
            out_ref[pl.ds(row0 + c * _CHUNK, _CHUNK), :] = jnp.sum(
                t * v_ref[...][None, :, :], axis=2)
        pl.when(i < n_blk - (_DEPTH - 1))(lambda: start(i + _DEPTH - 1))
        return ()

    jax.lax.fori_loop(0, n_blk, body, ())


def _softmax_t_kernel(s_ref, out_ref):
    s = s_ref[...]
    m = jnp.max(s, axis=0, keepdims=True)
    e = jnp.exp(s - m)
    p = e * (1.0 / jnp.sum(e, axis=0, keepdims=True))
    out_ref[...] = jnp.transpose(p)[:, None, :]


def _luong_concat(hidden, outputs, w, b, v, *, interpret=False):
    S, B, H = outputs.shape
    hp = jax.lax.Precision.HIGHEST

    hidden_bm = hidden.reshape(B, H).astype(jnp.float32)
    w = w.astype(jnp.float32)
    hproj = jnp.dot(hidden_bm, jnp.transpose(w[:, :H]), precision=hp) + b[None, :]
    w_o_t = jnp.transpose(w[:, H:]).astype(jnp.bfloat16)
    v2 = v.astype(jnp.float32).reshape(1, H)

    s_half = S // 2

    def rep(shape):
        return pl.BlockSpec(shape, lambda s: (0,) * len(shape))

    flops = 2 * S * B * H * H
    cost = pl.CostEstimate(flops=flops, transcendentals=S * B * H,
                           bytes_accessed=S * B * H * 4 + S * B * 4)

    scores = pl.pallas_call(
        functools.partial(_score_kernel, s_half=s_half),
        out_shape=jax.ShapeDtypeStruct((S, B), jnp.float32),
        grid=(2,),
        in_specs=[rep((B, H)), rep((H, H)), rep((1, H)),
                  pl.BlockSpec(memory_space=pl.ANY)],
        out_specs=pl.BlockSpec((s_half, B), lambda s: (s, 0)),
        scratch_shapes=[
            pltpu.VMEM((_DEPTH, _BLK, B, H), jnp.float32),
            pltpu.SemaphoreType.DMA((_DEPTH,)),
        ],
        compiler_params=pltpu.CompilerParams(
            dimension_semantics=("parallel",),
            vmem_limit_bytes=60 * 1024 * 1024),
        cost_estimate=cost,
        interpret=interpret,
    )(hproj, w_o_t, v2, outputs)

    vmem = pl.BlockSpec(memory_space=pltpu.MemorySpace.VMEM)
    return pl.pallas_call(
        _softmax_t_kernel,
        out_shape=jax.ShapeDtypeStruct((B, 1, S), jnp.float32),
        in_specs=[vmem],
        out_specs=vmem,
        interpret=interpret,
    )(scores)


def kernel(hidden, outputs, attention_w, attention_b, attention_v):
    return _luong_concat(hidden, outputs, attention_w, attention_b,
                         attention_v)


# speedup vs baseline: 1.1433x; 1.1433x over previous
import functools

import jax
import jax.numpy as jnp
from jax.experimental import pallas as pl
from jax.experimental.pallas import tpu as pltpu


def _score_kernel(hproj_ref, w_ref, v_ref, o_ref, out_ref, *, chunk_s):
    st = o_ref.shape[0]
    for c in range(st // chunk_s):
        o = o_ref[pl.ds(c * chunk_s, chunk_s), :, :]
        cs, b, h = o.shape
        lhs = o.reshape(cs * b, h).astype(jnp.bfloat16)
        oproj = jnp.dot(lhs, w_ref[...],
                        preferred_element_type=jnp.float32).reshape(cs, b, h)
        t = jnp.tanh(oproj + hproj_ref[...][None, :, :])
        out_ref[pl.ds(c * chunk_s, chunk_s), :] = jnp.sum(
            t * v_ref[...][None, :, :], axis=2)


def _softmax_t_kernel(s_ref, out_ref):
    s = s_ref[...]
    m = jnp.max(s, axis=0, keepdims=True)
    e = jnp.exp(s - m)
    p = e * (1.0 / jnp.sum(e, axis=0, keepdims=True))
    out_ref[...] = jnp.transpose(p)[:, None, :]


def _luong_concat(hidden, outputs, w, b, v, *, interpret=False):
    S, B, H = outputs.shape
    hp = jax.lax.Precision.HIGHEST

    hidden_bm = hidden.reshape(B, H).astype(jnp.float32)
    w = w.astype(jnp.float32)
    hproj = jnp.dot(hidden_bm, jnp.transpose(w[:, :H]), precision=hp) + b[None, :]
    w_o_t = jnp.transpose(w[:, H:]).astype(jnp.bfloat16)
    v2 = v.astype(jnp.float32).reshape(1, H)

    st = 128
    chunk_s = 8
    n_tiles = pl.cdiv(S, st)

    def rep(shape):
        return pl.BlockSpec(shape, lambda s: (0,) * len(shape))

    flops = 2 * S * B * H * H
    cost = pl.CostEstimate(flops=flops, transcendentals=S * B * H,
                           bytes_accessed=S * B * H * 4 + S * B * 4)

    scores = pl.pallas_call(
        functools.partial(_score_kernel, chunk_s=chunk_s),
        out_shape=jax.ShapeDtypeStruct((S, B), jnp.float32),
        grid=(n_tiles,),
        in_specs=[rep((B, H)), rep((H, H)), rep((1, H)),
                  pl.BlockSpec((st, B, H), lambda s: (s, 0, 0))],
        out_specs=pl.BlockSpec((st, B), lambda s: (s, 0)),
        compiler_params=pltpu.CompilerParams(
            dimension_semantics=("parallel",),
            vmem_limit_bytes=60 * 1024 * 1024),
        cost_estimate=cost,
        interpret=interpret,
    )(hproj, w_o_t, v2, outputs)

    vmem = pl.BlockSpec(memory_space=pltpu.MemorySpace.VMEM)
    return pl.pallas_call(
        _softmax_t_kernel,
        out_shape=jax.ShapeDtypeStruct((B, 1, S), jnp.float32),
        in_specs=[vmem],
        out_specs=vmem,
        interpret=interpret,
    )(scores)


def kernel(hidden, outputs, attention_w, attention_b, attention_v):
    return _luong_concat(hidden, outputs, attention_w, attention_b,
                         attention_v)
